# baseline (speedup 1.0000x reference)
import functools

import jax
import jax.numpy as jnp
from jax import lax
from jax.experimental import pallas as pl
from jax.experimental.pallas import tpu as pltpu
from jax.experimental.pallas import tpu_sc as plsc

DIM = 64
EPS = 1e-5

NUM_CORES = 2
NUM_SUBCORES = 16
NUM_WORKERS = NUM_CORES * NUM_SUBCORES
LANES = 16

SUBG = 128
NSUB = 4
CHUNK = SUBG * NSUB


ROW_UNROLL = 8


def _ln_chunk(rows_ref, wb_v, lane):
    del lane
    wq = [wb_v[0, pl.ds(q * 16, 16)] for q in range(4)]
    bq = [wb_v[1, pl.ds(q * 16, 16)] for q in range(4)]

    @plsc.parallel_loop(0, CHUNK, 1, unroll=ROW_UNROLL)
    def row_body(row):
        v = [rows_ref[row, pl.ds(q * 16, 16)] for q in range(4)]
        s = (v[0] + v[1]) + (v[2] + v[3])
        sq = (v[0] * v[0] + v[1] * v[1]) + (v[2] * v[2] + v[3] * v[3])
        mu_s = jnp.sum(s) * (1.0 / DIM)
        var_s = jnp.sum(sq) * (1.0 / DIM) - mu_s * mu_s
        x_s = var_s + EPS
        i_s = lax.bitcast_convert_type(x_s, jnp.int32)
        y_s = lax.bitcast_convert_type(
            jnp.int32(0x5F3759DF) - (i_s >> 1), jnp.float32
        )
        for _ in range(3):
            y_s = y_s * (1.5 - 0.5 * x_s * y_s * y_s)
        mu_b = jnp.broadcast_to(mu_s, (16,))
        y_b = jnp.broadcast_to(y_s, (16,))
        for q in range(4):
            rows_ref[row, pl.ds(q * 16, 16)] = (
                (v[q] - mu_b) * y_b * wq[q] + bq[q]
            )


def _sc_body(
    table_h, x_h, w_h, b_h, out_h,
    idx_v, rows_v, wb_v,
    sem_i0, sem_i1, sem_g0, sem_g1, sem_o0, sem_o1,
):
    n_rows = x_h.shape[0]
    rows_per_w = n_rows // NUM_WORKERS
    n_chunks = rows_per_w // CHUNK
    sem_i = (sem_i0, sem_i1)
    sem_g = (sem_g0, sem_g1)
    sem_o = (sem_o0, sem_o1)

    wid = lax.axis_index("s") * NUM_CORES + lax.axis_index("c")
    base_w = wid * rows_per_w
    pltpu.sync_copy(w_h, wb_v.at[0])
    pltpu.sync_copy(b_h, wb_v.at[1])

    lane = lax.iota(jnp.int32, 16)

    def start_idx(g, b):
        base = base_w + g * CHUNK
        for j in range(NSUB):
            pltpu.async_copy(
                x_h.at[pl.ds(base + j * SUBG, SUBG)],
                idx_v.at[b].at[j],
                sem_i[b],
            )

    def wait_idx(b):
        for j in range(NSUB):
            pltpu.make_async_copy(
                x_h.at[pl.ds(0, SUBG)], idx_v.at[b].at[j], sem_i[b]
            ).wait()

    def start_gather(b):
        for j in range(NSUB):
            pltpu.async_copy(
                table_h.at[idx_v.at[b].at[j]],
                rows_v.at[b].at[pl.ds(j * SUBG, SUBG)],
                sem_g[b],
            )

    def wait_gather(b):
        pltpu.make_async_copy(
            table_h.at[pl.ds(0, CHUNK)], rows_v.at[b], sem_g[b]
        ).wait()

    def start_out(g, b):
        base = base_w + g * CHUNK
        pltpu.async_copy(
            rows_v.at[b],
            out_h.at[pl.ds(base, CHUNK), pl.ds(0, DIM)],
            sem_o[b],
        )

    def wait_out(b):
        pltpu.make_async_copy(
            rows_v.at[b], out_h.at[pl.ds(0, CHUNK), pl.ds(0, DIM)], sem_o[b]
        ).wait()

    start_idx(0, 0)
    wait_idx(0)
    start_gather(0)
    start_idx(1, 1)

    def pair_body(jj, carry):
        for b in range(2):
            g = jj * 2 + b
            nb = 1 - b
            wait_gather(b)

            @pl.when(g + 1 < n_chunks)
            def _():
                wait_idx(nb)
                start_gather(nb)

            @pl.when(g + 2 < n_chunks)
            def _():
                start_idx(g + 2, b)

            @pl.when(g >= 2)
            def _():
                wait_out(b)

            _ln_chunk(rows_v.at[b], wb_v, lane)
            start_out(g, b)
        return carry

    lax.fori_loop(0, n_chunks // 2, pair_body, 0)
    wait_out(0)
    wait_out(1)


def _make_sc_call(n_rows):
    mesh = plsc.VectorSubcoreMesh(
        core_axis_name="c",
        subcore_axis_name="s",
        num_cores=NUM_CORES,
        num_subcores=NUM_SUBCORES,
    )
    return pl.kernel(
        _sc_body,
        out_type=jax.ShapeDtypeStruct((n_rows, 128), jnp.float32),
        mesh=mesh,
        compiler_params=pltpu.CompilerParams(
            needs_layout_passes=False, use_tc_tiling_on_sc=False
        ),
        scratch_types=[
            pltpu.VMEM((2, NSUB, SUBG), jnp.int32),
            pltpu.VMEM((2, CHUNK, DIM), jnp.float32),
            pltpu.VMEM((2, DIM), jnp.float32),
            pltpu.SemaphoreType.DMA,
            pltpu.SemaphoreType.DMA,
            pltpu.SemaphoreType.DMA,
            pltpu.SemaphoreType.DMA,
            pltpu.SemaphoreType.DMA,
            pltpu.SemaphoreType.DMA,
        ],
    )


BPB = 8


def _finish_body(raw_ref, x_ref, o_ref, m_ref):
    o_ref[...] = raw_ref[:, :DIM].reshape(BPB, 200, DIM)
    m_ref[...] = x_ref[...] > 0


@jax.jit
def kernel(x, table, ln_weight, ln_bias):
    batches, seq = x.shape
    xf = x.reshape(-1).astype(jnp.int32)
    raw = _make_sc_call(xf.shape[0])(table, xf, ln_weight, ln_bias)
    normed, mask = pl.pallas_call(
        _finish_body,
        grid=(batches // BPB,),
        in_specs=[
            pl.BlockSpec((BPB * seq, 128), lambda i: (i, 0)),
            pl.BlockSpec((BPB, seq), lambda i: (i, 0)),
        ],
        out_specs=[
            pl.BlockSpec((BPB, seq, DIM), lambda i: (i, 0, 0)),
            pl.BlockSpec((BPB, seq), lambda i: (i, 0)),
        ],
        out_shape=[
            jax.ShapeDtypeStruct((batches, seq, DIM), jnp.float32),
            jax.ShapeDtypeStruct((batches, seq), jnp.bool_),
        ],
    )(raw, x)
    return normed, mask


# speedup vs baseline: 2.1342x; 2.1342x over previous
import functools

import jax
import jax.numpy as jnp
from jax import lax
from jax.experimental import pallas as pl
from jax.experimental.pallas import tpu as pltpu
from jax.experimental.pallas import tpu_sc as plsc

DIM = 64
EPS = 1e-5

NUM_CORES = 2
NUM_SUBCORES = 16
NUM_WORKERS = NUM_CORES * NUM_SUBCORES
LANES = 16

SUBG = 128
NSUB = 4
CHUNK = SUBG * NSUB


ROW_UNROLL = 8


def _ln_chunk(rows_ref, wb_v, lane):
    del lane
    wq = [wb_v[0, pl.ds(q * 16, 16)] for q in range(4)]
    bq = [wb_v[1, pl.ds(q * 16, 16)] for q in range(4)]

    @plsc.parallel_loop(0, CHUNK, 1, unroll=ROW_UNROLL)
    def row_body(row):
        v = [rows_ref[row, pl.ds(q * 16, 16)] for q in range(4)]
        s = (v[0] + v[1]) + (v[2] + v[3])
        sq = (v[0] * v[0] + v[1] * v[1]) + (v[2] * v[2] + v[3] * v[3])
        mu_s = jnp.sum(s) * (1.0 / DIM)
        var_s = jnp.sum(sq) * (1.0 / DIM) - mu_s * mu_s
        x_s = var_s + EPS
        i_s = lax.bitcast_convert_type(x_s, jnp.int32)
        y_s = lax.bitcast_convert_type(
            jnp.int32(0x5F3759DF) - (i_s >> 1), jnp.float32
        )
        for _ in range(3):
            y_s = y_s * (1.5 - 0.5 * x_s * y_s * y_s)
        mu_b = jnp.broadcast_to(mu_s, (16,))
        y_b = jnp.broadcast_to(y_s, (16,))
        for q in range(4):
            rows_ref[row, pl.ds(q * 16, 16)] = (
                (v[q] - mu_b) * y_b * wq[q] + bq[q]
            )


def _sc_body(
    table_h, x_h, w_h, b_h, out_h,
    idx_v, rows_v, wb_v,
    sem_i0, sem_i1, sem_g0, sem_g1, sem_o0, sem_o1,
):
    n_rows = x_h.shape[0]
    rows_per_w = n_rows // NUM_WORKERS
    n_chunks = rows_per_w // CHUNK
    sem_i = (sem_i0, sem_i1)
    sem_g = (sem_g0, sem_g1)
    sem_o = (sem_o0, sem_o1)

    wid = lax.axis_index("s") * NUM_CORES + lax.axis_index("c")
    base_w = wid * rows_per_w
    pltpu.sync_copy(w_h, wb_v.at[0])
    pltpu.sync_copy(b_h, wb_v.at[1])

    lane = lax.iota(jnp.int32, 16)

    def start_idx(g, b):
        base = base_w + g * CHUNK
        for j in range(NSUB):
            pltpu.async_copy(
                x_h.at[pl.ds(base + j * SUBG, SUBG)],
                idx_v.at[b].at[j],
                sem_i[b],
            )

    def wait_idx(b):
        for j in range(NSUB):
            pltpu.make_async_copy(
                x_h.at[pl.ds(0, SUBG)], idx_v.at[b].at[j], sem_i[b]
            ).wait()

    def start_gather(b):
        for j in range(NSUB):
            pltpu.async_copy(
                table_h.at[idx_v.at[b].at[j]],
                rows_v.at[b].at[pl.ds(j * SUBG, SUBG)],
                sem_g[b],
            )

    def wait_gather(b):
        pltpu.make_async_copy(
            table_h.at[pl.ds(0, CHUNK)], rows_v.at[b], sem_g[b]
        ).wait()

    def start_out(g, b):
        base = base_w + g * CHUNK
        pltpu.async_copy(
            rows_v.at[b],
            out_h.at[pl.ds(base, CHUNK), pl.ds(0, DIM)],
            sem_o[b],
        )

    def wait_out(b):
        pltpu.make_async_copy(
            rows_v.at[b], out_h.at[pl.ds(0, CHUNK), pl.ds(0, DIM)], sem_o[b]
        ).wait()

    start_idx(0, 0)
    wait_idx(0)
    start_gather(0)
    start_idx(1, 1)

    def pair_body(jj, carry):
        for b in range(2):
            g = jj * 2 + b
            nb = 1 - b
            wait_gather(b)

            @pl.when(g + 1 < n_chunks)
            def _():
                wait_idx(nb)
                start_gather(nb)

            @pl.when(g + 2 < n_chunks)
            def _():
                start_idx(g + 2, b)

            @pl.when(g >= 2)
            def _():
                wait_out(b)

            _ln_chunk(rows_v.at[b], wb_v, lane)
            start_out(g, b)
        return carry

    lax.fori_loop(0, n_chunks // 2, pair_body, 0)
    wait_out(0)
    wait_out(1)


def _make_sc_call(n_rows):
    mesh = plsc.VectorSubcoreMesh(
        core_axis_name="c",
        subcore_axis_name="s",
        num_cores=NUM_CORES,
        num_subcores=NUM_SUBCORES,
    )
    return pl.kernel(
        _sc_body,
        out_type=jax.ShapeDtypeStruct((n_rows, 128), jnp.float32),
        mesh=mesh,
        compiler_params=pltpu.CompilerParams(
            needs_layout_passes=False, use_tc_tiling_on_sc=False
        ),
        scratch_types=[
            pltpu.VMEM((2, NSUB, SUBG), jnp.int32),
            pltpu.VMEM((2, CHUNK, DIM), jnp.float32),
            pltpu.VMEM((2, DIM), jnp.float32),
            pltpu.SemaphoreType.DMA,
            pltpu.SemaphoreType.DMA,
            pltpu.SemaphoreType.DMA,
            pltpu.SemaphoreType.DMA,
            pltpu.SemaphoreType.DMA,
            pltpu.SemaphoreType.DMA,
        ],
    )


def _mask_body(x_ref, o_ref):
    o_ref[...] = x_ref[...] > 0


@jax.jit
def kernel(x, table, ln_weight, ln_bias):
    batches, seq = x.shape
    xf = x.reshape(-1).astype(jnp.int32)
    raw = _make_sc_call(xf.shape[0])(table, xf, ln_weight, ln_bias)
    normed = raw.reshape(batches, seq, 128)[:, :, :DIM]
    mask = pl.pallas_call(
        _mask_body,
        out_shape=jax.ShapeDtypeStruct(x.shape, jnp.bool_),
    )(x)
    return normed, mask
